# Initial kernel scaffold; baseline (speedup 1.0000x reference)
import jax
import jax.numpy as jnp
from jax.experimental import pallas as pl


def kernel(x, l, W_p1, b_p1, W_p2, b_p2, W_r1, b_r1, W_r2, b_r2):
    raise NotImplementedError("write your pallas kernel here")


# speedup vs baseline: 3.0607x; 3.0607x over previous
import functools

import jax
import jax.numpy as jnp
from jax import lax
from jax.experimental import pallas as pl
from jax.experimental.pallas import tpu as pltpu
from jax.experimental.pallas import tpu_sc as plsc

N = 320000
D_IN = 128
D_H = 64
S = 10000

NC, NS = 2, 16
NW = NC * NS
SEG_PER_W = 320
SP = SEG_PER_W * NW
CHUNK = 512
NB16 = N // 16

_DN = (((1,), (1,)), ((), ()))


def _phi_body(x_ref, w1_ref, b1_ref, w2_ref, b2_ref, h_ref):
    x = x_ref[...]
    h1 = lax.dot_general(x, w1_ref[...], _DN, preferred_element_type=jnp.float32)
    h1 = jnp.maximum(h1 + b1_ref[...], 0.0)
    h2 = lax.dot_general(h1, w2_ref[...], _DN, preferred_element_type=jnp.float32)
    h_ref[...] = jnp.maximum(h2 + b2_ref[...], 0.0)


def _phi(x, w1, b1, w2, b2):
    blk = 2560
    grid = (N // blk,)
    return pl.pallas_call(
        _phi_body,
        grid=grid,
        in_specs=[
            pl.BlockSpec((blk, D_IN), lambda i: (i, 0)),
            pl.BlockSpec((D_IN, D_IN), lambda i: (0, 0)),
            pl.BlockSpec((1, D_IN), lambda i: (0, 0)),
            pl.BlockSpec((D_H, D_IN), lambda i: (0, 0)),
            pl.BlockSpec((1, D_H), lambda i: (0, 0)),
        ],
        out_specs=pl.BlockSpec((blk, D_H), lambda i: (i, 0)),
        out_shape=jax.ShapeDtypeStruct((N, D_H), jnp.float32),
        compiler_params=pltpu.CompilerParams(dimension_semantics=("parallel",)),
    )(x, w1, b1, w2, b2)


def _seg_body(ids_hbm, h_hbm, sum_hbm, mean_hbm, max_hbm,
              sum_v, mean_v, max_v, ids_v, h_v, bs_v):
    c = lax.axis_index("c")
    s = lax.axis_index("s")
    wid = s * NC + c
    lo_seg = wid * SEG_PER_W

    z16 = jnp.zeros((16,), jnp.float32)

    def zrow(i, carry):
        sum_v[pl.ds(i * 16, 16)] = z16
        mean_v[pl.ds(i * 16, 16)] = z16
        max_v[pl.ds(i * 16, 16)] = z16
        return carry

    lax.fori_loop(0, SEG_PER_W * 4, zrow, 0)

    def lower_bound(v):
        def body(_, state):
            lo, hi = state
            done = lo >= hi
            mid = jnp.minimum((lo + hi) // 2, NB16 - 1)
            pltpu.sync_copy(ids_hbm.at[pl.ds(mid * 16, 16)], bs_v)
            first = bs_v[...][0]
            ge = first >= v
            new_lo = jnp.where(done, lo, jnp.where(ge, lo, mid + 1))
            new_hi = jnp.where(done, hi, jnp.where(ge, mid, hi))
            return (new_lo, new_hi)

        fb, _ = lax.fori_loop(0, 15, body, (jnp.int32(0), jnp.int32(NB16)))
        blk = jnp.maximum(fb - 1, 0)
        pltpu.sync_copy(ids_hbm.at[pl.ds(blk * 16, 16)], bs_v)
        vec = bs_v[...]
        cnt_lt = jnp.int32(0)
        for j in range(16):
            cnt_lt = cnt_lt + jnp.where(vec[j] < v, 1, 0)
        return jnp.where(fb == 0, 0, (fb - 1) * 16 + cnt_lt)

    r0 = lower_bound(lo_seg)
    r1 = lower_bound(lo_seg + SEG_PER_W)
    base = (r0 // 8) * 8
    nchunks = (r1 - base + CHUNK - 1) // CHUNK

    def flush(cur_id, cnt, a, m):
        rel = cur_id - lo_seg
        for j in range(4):
            sum_v[pl.ds(rel * D_H + j * 16, 16)] = a[j]
            mean_v[pl.ds(rel * D_H + j * 16, 16)] = a[j] / cnt
            max_v[pl.ds(rel * D_H + j * 16, 16)] = m[j]

    def chunk_body(ci, carry):
        start = base + ci * CHUNK
        pltpu.sync_copy(ids_hbm.at[pl.ds(start, CHUNK)], ids_v.at[pl.ds(0, CHUNK)])
        pltpu.sync_copy(h_hbm.at[pl.ds(start * D_H, CHUNK * D_H)], h_v)
        i_lo = jnp.maximum(r0 - start, 0)
        i_hi = jnp.minimum(r1 - start, CHUNK)

        def row_body(i, rc):
            cur_id, cnt, a0, a1, a2, a3, m0, m1, m2, m3 = rc
            idv = ids_v[pl.ds(i, 16)][0]
            v0 = h_v[pl.ds(i * D_H, 16)]
            v1 = h_v[pl.ds(i * D_H + 16, 16)]
            v2 = h_v[pl.ds(i * D_H + 32, 16)]
            v3 = h_v[pl.ds(i * D_H + 48, 16)]
            is_new = idv != cur_id

            @pl.when(is_new & (cur_id >= 0))
            def _():
                flush(cur_id, cnt, (a0, a1, a2, a3), (m0, m1, m2, m3))

            a0 = jnp.where(is_new, v0, a0 + v0)
            a1 = jnp.where(is_new, v1, a1 + v1)
            a2 = jnp.where(is_new, v2, a2 + v2)
            a3 = jnp.where(is_new, v3, a3 + v3)
            m0 = jnp.where(is_new, v0, jnp.maximum(m0, v0))
            m1 = jnp.where(is_new, v1, jnp.maximum(m1, v1))
            m2 = jnp.where(is_new, v2, jnp.maximum(m2, v2))
            m3 = jnp.where(is_new, v3, jnp.maximum(m3, v3))
            cnt = jnp.where(is_new, 1.0, cnt + 1.0)
            return (idv, cnt, a0, a1, a2, a3, m0, m1, m2, m3)

        return lax.fori_loop(i_lo, i_hi, row_body, carry)

    init = (jnp.int32(-1), jnp.float32(0.0), z16, z16, z16, z16, z16, z16, z16, z16)
    fin = lax.fori_loop(0, nchunks, chunk_body, init)
    cur_id, cnt = fin[0], fin[1]

    @pl.when(cur_id >= 0)
    def _():
        flush(cur_id, cnt, fin[2:6], fin[6:10])

    pltpu.sync_copy(sum_v, sum_hbm.at[pl.ds(lo_seg * D_H, SEG_PER_W * D_H)])
    pltpu.sync_copy(mean_v, mean_hbm.at[pl.ds(lo_seg * D_H, SEG_PER_W * D_H)])
    pltpu.sync_copy(max_v, max_hbm.at[pl.ds(lo_seg * D_H, SEG_PER_W * D_H)])


def _segstats(ids, h_flat):
    mesh = plsc.VectorSubcoreMesh(core_axis_name="c", subcore_axis_name="s",
                                  num_cores=NC, num_subcores=NS)
    f = pl.kernel(
        _seg_body,
        out_type=[jax.ShapeDtypeStruct((SP * D_H,), jnp.float32)] * 3,
        mesh=mesh,
        scratch_types=[
            pltpu.VMEM((SEG_PER_W * D_H,), jnp.float32),
            pltpu.VMEM((SEG_PER_W * D_H,), jnp.float32),
            pltpu.VMEM((SEG_PER_W * D_H,), jnp.float32),
            pltpu.VMEM((CHUNK + 16,), jnp.int32),
            pltpu.VMEM((CHUNK * D_H,), jnp.float32),
            pltpu.VMEM((16,), jnp.int32),
        ],
    )
    return f(ids, h_flat)


def _rho_body(s_ref, mn_ref, mx_ref, a_ref, b_ref, c_ref, br1_ref, w2_ref,
              br2_ref, o_ref):
    y = lax.dot_general(s_ref[...], a_ref[...], _DN, preferred_element_type=jnp.float32)
    y = y + lax.dot_general(mn_ref[...], b_ref[...], _DN, preferred_element_type=jnp.float32)
    y = y + lax.dot_general(mx_ref[...], c_ref[...], _DN, preferred_element_type=jnp.float32)
    y = y + br1_ref[...]
    z = lax.dot_general(y, w2_ref[...], _DN, preferred_element_type=jnp.float32)
    z = z + br2_ref[...]
    zmax = jnp.max(z, axis=1, keepdims=True)
    zs = z - zmax
    lse = jnp.log(jnp.sum(jnp.exp(zs), axis=1, keepdims=True))
    o_ref[...] = zs - lse


def _rho(xs, xmn, xmx, wa, wb, wc, br1, w2, br2):
    blk = 1000
    grid = (S // blk,)
    return pl.pallas_call(
        _rho_body,
        grid=grid,
        in_specs=[
            pl.BlockSpec((blk, D_H), lambda i: (i, 0)),
            pl.BlockSpec((blk, D_H), lambda i: (i, 0)),
            pl.BlockSpec((blk, D_H), lambda i: (i, 0)),
            pl.BlockSpec((D_H, D_H), lambda i: (0, 0)),
            pl.BlockSpec((D_H, D_H), lambda i: (0, 0)),
            pl.BlockSpec((D_H, D_H), lambda i: (0, 0)),
            pl.BlockSpec((1, D_H), lambda i: (0, 0)),
            pl.BlockSpec((10, D_H), lambda i: (0, 0)),
            pl.BlockSpec((1, 10), lambda i: (0, 0)),
        ],
        out_specs=pl.BlockSpec((blk, 10), lambda i: (i, 0)),
        out_shape=jax.ShapeDtypeStruct((S, 10), jnp.float32),
        compiler_params=pltpu.CompilerParams(dimension_semantics=("parallel",)),
    )(xs, xmn, xmx, wa, wb, wc, br1, w2, br2)


def kernel(x, l, W_p1, b_p1, W_p2, b_p2, W_r1, b_r1, W_r2, b_r2):
    ids = l[:, 0].astype(jnp.int32)
    h = _phi(x, W_p1, b_p1.reshape(1, -1), W_p2, b_p2.reshape(1, -1))
    sums, means, maxs = _segstats(ids, h.reshape(-1))
    sums = sums.reshape(SP, D_H)[:S]
    means = means.reshape(SP, D_H)[:S]
    maxs = maxs.reshape(SP, D_H)[:S]
    out = _rho(sums, means, maxs,
               W_r1[:, 0:64], W_r1[:, 64:128], W_r1[:, 128:192],
               b_r1.reshape(1, -1), W_r2, b_r2.reshape(1, -1))
    return out


# speedup vs baseline: 3.7842x; 1.2364x over previous
import functools

import jax
import jax.numpy as jnp
from jax import lax
from jax.experimental import pallas as pl
from jax.experimental.pallas import tpu as pltpu
from jax.experimental.pallas import tpu_sc as plsc

N = 320000
D_IN = 128
D_H = 64
S = 10000

NC, NS = 2, 16
NW = NC * NS
SEG_PER_W = 320
SP = SEG_PER_W * NW
CHUNK = 512
NB16 = N // 16

_DN = (((1,), (1,)), ((), ()))


def _phi_body(x_ref, w1_ref, b1_ref, w2_ref, b2_ref, h_ref):
    x = x_ref[...]
    h1 = lax.dot_general(x, w1_ref[...], _DN, preferred_element_type=jnp.float32)
    h1 = jnp.maximum(h1 + b1_ref[...], 0.0)
    h2 = lax.dot_general(h1, w2_ref[...], _DN, preferred_element_type=jnp.float32)
    h_ref[...] = jnp.maximum(h2 + b2_ref[...], 0.0)


def _phi(x, w1, b1, w2, b2):
    blk = 2560
    grid = (N // blk,)
    return pl.pallas_call(
        _phi_body,
        grid=grid,
        in_specs=[
            pl.BlockSpec((blk, D_IN), lambda i: (i, 0)),
            pl.BlockSpec((D_IN, D_IN), lambda i: (0, 0)),
            pl.BlockSpec((1, D_IN), lambda i: (0, 0)),
            pl.BlockSpec((D_H, D_IN), lambda i: (0, 0)),
            pl.BlockSpec((1, D_H), lambda i: (0, 0)),
        ],
        out_specs=pl.BlockSpec((blk, D_H), lambda i: (i, 0)),
        out_shape=jax.ShapeDtypeStruct((N, D_H), jnp.float32),
        compiler_params=pltpu.CompilerParams(dimension_semantics=("parallel",)),
    )(x, w1, b1, w2, b2)


def _seg_body(ids_hbm, h_hbm, sum_hbm, mean_hbm, max_hbm,
              sum_v, mean_v, max_v, ids_v, h_v, bs_v):
    c = lax.axis_index("c")
    s = lax.axis_index("s")
    wid = s * NC + c
    lo_seg = wid * SEG_PER_W

    z16 = jnp.zeros((16,), jnp.float32)

    def zrow(i, carry):
        sum_v[pl.ds(i * 16, 16)] = z16
        mean_v[pl.ds(i * 16, 16)] = z16
        max_v[pl.ds(i * 16, 16)] = z16
        return carry

    lax.fori_loop(0, SEG_PER_W * 4, zrow, 0)

    def lower_bound(v):
        def body(_, state):
            lo, hi = state
            done = lo >= hi
            mid = jnp.minimum((lo + hi) // 2, NB16 - 1)
            pltpu.sync_copy(ids_hbm.at[pl.ds(mid * 16, 16)], bs_v)
            first = bs_v[...][0]
            ge = first >= v
            new_lo = jnp.where(done, lo, jnp.where(ge, lo, mid + 1))
            new_hi = jnp.where(done, hi, jnp.where(ge, mid, hi))
            return (new_lo, new_hi)

        fb, _ = lax.fori_loop(0, 15, body, (jnp.int32(0), jnp.int32(NB16)))
        blk = jnp.maximum(fb - 1, 0)
        pltpu.sync_copy(ids_hbm.at[pl.ds(blk * 16, 16)], bs_v)
        vec = bs_v[...]
        cnt_lt = jnp.int32(0)
        for j in range(16):
            cnt_lt = cnt_lt + jnp.where(vec[j] < v, 1, 0)
        return jnp.where(fb == 0, 0, (fb - 1) * 16 + cnt_lt)

    r0 = lower_bound(lo_seg)
    r1 = lower_bound(lo_seg + SEG_PER_W)
    base = (r0 // CHUNK) * CHUNK
    nchunks = (r1 - base + CHUNK - 1) // CHUNK

    def flush(rel, cnt, a, m):
        for j in range(4):
            sum_v[pl.ds(rel * D_H + j * 16, 16)] = a[j]
            mean_v[pl.ds(rel * D_H + j * 16, 16)] = a[j] / cnt
            max_v[pl.ds(rel * D_H + j * 16, 16)] = m[j]

    def chunk_body(ci, carry):
        start = base + ci * CHUNK
        pltpu.sync_copy(ids_hbm.at[pl.ds(start, CHUNK)], ids_v)
        pltpu.sync_copy(h_hbm.at[pl.ds(start * D_H, CHUNK * D_H)], h_v)

        def row_step(i, idv, rc):
            cur_rel, cnt, a0, a1, a2, a3, m0, m1, m2, m3 = rc
            rel = idv - lo_seg
            valid = (rel >= 0) & (rel < SEG_PER_W)
            rel = jnp.where(valid, rel, SEG_PER_W)
            v0 = h_v[pl.ds(i * D_H, 16)]
            v1 = h_v[pl.ds(i * D_H + 16, 16)]
            v2 = h_v[pl.ds(i * D_H + 32, 16)]
            v3 = h_v[pl.ds(i * D_H + 48, 16)]
            is_new = rel != cur_rel

            @pl.when(is_new & (cur_rel >= 0))
            def _():
                flush(cur_rel, cnt, (a0, a1, a2, a3), (m0, m1, m2, m3))

            a0 = jnp.where(is_new, v0, a0 + v0)
            a1 = jnp.where(is_new, v1, a1 + v1)
            a2 = jnp.where(is_new, v2, a2 + v2)
            a3 = jnp.where(is_new, v3, a3 + v3)
            m0 = jnp.where(is_new, v0, jnp.maximum(m0, v0))
            m1 = jnp.where(is_new, v1, jnp.maximum(m1, v1))
            m2 = jnp.where(is_new, v2, jnp.maximum(m2, v2))
            m3 = jnp.where(is_new, v3, jnp.maximum(m3, v3))
            cnt = jnp.where(is_new, 1.0, cnt + 1.0)
            return (rel, cnt, a0, a1, a2, a3, m0, m1, m2, m3)

        def group_body(g, rc):
            idvec = ids_v[pl.ds(g * 16, 16)]
            for j in range(16):
                rc = row_step(g * 16 + j, idvec[j], rc)
            return rc

        return lax.fori_loop(0, CHUNK // 16, group_body, carry)

    init = (jnp.int32(-1), jnp.float32(0.0), z16, z16, z16, z16, z16, z16, z16, z16)
    fin = lax.fori_loop(0, nchunks, chunk_body, init)
    cur_rel, cnt = fin[0], fin[1]

    @pl.when(cur_rel >= 0)
    def _():
        flush(cur_rel, cnt, fin[2:6], fin[6:10])

    pltpu.sync_copy(sum_v.at[pl.ds(0, SEG_PER_W * D_H)],
                    sum_hbm.at[pl.ds(lo_seg * D_H, SEG_PER_W * D_H)])
    pltpu.sync_copy(mean_v.at[pl.ds(0, SEG_PER_W * D_H)],
                    mean_hbm.at[pl.ds(lo_seg * D_H, SEG_PER_W * D_H)])
    pltpu.sync_copy(max_v.at[pl.ds(0, SEG_PER_W * D_H)],
                    max_hbm.at[pl.ds(lo_seg * D_H, SEG_PER_W * D_H)])


def _segstats(ids, h_flat):
    mesh = plsc.VectorSubcoreMesh(core_axis_name="c", subcore_axis_name="s",
                                  num_cores=NC, num_subcores=NS)
    f = pl.kernel(
        _seg_body,
        out_type=[jax.ShapeDtypeStruct((SP * D_H,), jnp.float32)] * 3,
        mesh=mesh,
        scratch_types=[
            pltpu.VMEM(((SEG_PER_W + 1) * D_H,), jnp.float32),
            pltpu.VMEM(((SEG_PER_W + 1) * D_H,), jnp.float32),
            pltpu.VMEM(((SEG_PER_W + 1) * D_H,), jnp.float32),
            pltpu.VMEM((CHUNK,), jnp.int32),
            pltpu.VMEM((CHUNK * D_H,), jnp.float32),
            pltpu.VMEM((16,), jnp.int32),
        ],
    )
    return f(ids, h_flat)


def _rho_body(s_ref, mn_ref, mx_ref, a_ref, b_ref, c_ref, br1_ref, w2_ref,
              br2_ref, o_ref):
    y = lax.dot_general(s_ref[...], a_ref[...], _DN, preferred_element_type=jnp.float32)
    y = y + lax.dot_general(mn_ref[...], b_ref[...], _DN, preferred_element_type=jnp.float32)
    y = y + lax.dot_general(mx_ref[...], c_ref[...], _DN, preferred_element_type=jnp.float32)
    y = y + br1_ref[...]
    z = lax.dot_general(y, w2_ref[...], _DN, preferred_element_type=jnp.float32)
    z = z + br2_ref[...]
    zmax = jnp.max(z, axis=1, keepdims=True)
    zs = z - zmax
    lse = jnp.log(jnp.sum(jnp.exp(zs), axis=1, keepdims=True))
    o_ref[...] = zs - lse


def _rho(xs, xmn, xmx, wa, wb, wc, br1, w2, br2):
    blk = 1000
    grid = (S // blk,)
    return pl.pallas_call(
        _rho_body,
        grid=grid,
        in_specs=[
            pl.BlockSpec((blk, D_H), lambda i: (i, 0)),
            pl.BlockSpec((blk, D_H), lambda i: (i, 0)),
            pl.BlockSpec((blk, D_H), lambda i: (i, 0)),
            pl.BlockSpec((D_H, D_H), lambda i: (0, 0)),
            pl.BlockSpec((D_H, D_H), lambda i: (0, 0)),
            pl.BlockSpec((D_H, D_H), lambda i: (0, 0)),
            pl.BlockSpec((1, D_H), lambda i: (0, 0)),
            pl.BlockSpec((10, D_H), lambda i: (0, 0)),
            pl.BlockSpec((1, 10), lambda i: (0, 0)),
        ],
        out_specs=pl.BlockSpec((blk, 10), lambda i: (i, 0)),
        out_shape=jax.ShapeDtypeStruct((S, 10), jnp.float32),
        compiler_params=pltpu.CompilerParams(dimension_semantics=("parallel",)),
    )(xs, xmn, xmx, wa, wb, wc, br1, w2, br2)


def kernel(x, l, W_p1, b_p1, W_p2, b_p2, W_r1, b_r1, W_r2, b_r2):
    ids = l[:, 0].astype(jnp.int32)
    h = _phi(x, W_p1, b_p1.reshape(1, -1), W_p2, b_p2.reshape(1, -1))
    sums, means, maxs = _segstats(ids, h.reshape(-1))
    sums = sums.reshape(SP, D_H)[:S]
    means = means.reshape(SP, D_H)[:S]
    maxs = maxs.reshape(SP, D_H)[:S]
    out = _rho(sums, means, maxs,
               W_r1[:, 0:64], W_r1[:, 64:128], W_r1[:, 128:192],
               b_r1.reshape(1, -1), W_r2, b_r2.reshape(1, -1))
    return out


# speedup vs baseline: 4.2001x; 1.1099x over previous
import functools

import jax
import jax.numpy as jnp
from jax import lax
from jax.experimental import pallas as pl
from jax.experimental.pallas import tpu as pltpu
from jax.experimental.pallas import tpu_sc as plsc

N = 320000
D_IN = 128
D_H = 64
S = 10000

NC, NS = 2, 16
NW = NC * NS
SEG_PER_W = 320
SP = SEG_PER_W * NW
CHUNK = 256
NB16 = N // 16

_DN = (((1,), (1,)), ((), ()))


def _phi_body(x_ref, w1_ref, b1_ref, w2_ref, b2_ref, h_ref):
    x = x_ref[...]
    h1 = lax.dot_general(x, w1_ref[...], _DN, preferred_element_type=jnp.float32)
    h1 = jnp.maximum(h1 + b1_ref[...], 0.0)
    h2 = lax.dot_general(h1, w2_ref[...], _DN, preferred_element_type=jnp.float32)
    h_ref[...] = jnp.maximum(h2 + b2_ref[...], 0.0)


def _phi(x, w1, b1, w2, b2):
    blk = 2560
    grid = (N // blk,)
    return pl.pallas_call(
        _phi_body,
        grid=grid,
        in_specs=[
            pl.BlockSpec((blk, D_IN), lambda i: (i, 0)),
            pl.BlockSpec((D_IN, D_IN), lambda i: (0, 0)),
            pl.BlockSpec((1, D_IN), lambda i: (0, 0)),
            pl.BlockSpec((D_H, D_IN), lambda i: (0, 0)),
            pl.BlockSpec((1, D_H), lambda i: (0, 0)),
        ],
        out_specs=pl.BlockSpec((blk, D_H), lambda i: (i, 0)),
        out_shape=jax.ShapeDtypeStruct((N, D_H), jnp.float32),
        compiler_params=pltpu.CompilerParams(dimension_semantics=("parallel",)),
    )(x, w1, b1, w2, b2)


def _seg_body(ids_hbm, h_hbm, sum_hbm, mean_hbm, max_hbm,
             sum_v, mean_v, max_v, ids_v0, ids_v1, h_v0, h_v1, bs_v,
             sem0, sem1):
    c = lax.axis_index("c")
    s = lax.axis_index("s")
    wid = s * NC + c
    lo_seg = wid * SEG_PER_W

    z16 = jnp.zeros((16,), jnp.float32)
    neg16 = jnp.full((16,), -1, jnp.int32)

    def zrow(i, carry):
        sum_v[pl.ds(i * 16, 16)] = z16
        mean_v[pl.ds(i * 16, 16)] = z16
        max_v[pl.ds(i * 16, 16)] = z16
        return carry

    lax.fori_loop(0, SEG_PER_W * 4, zrow, 0)

    def lower_bound(v):
        def body(_, state):
            lo, hi = state
            done = lo >= hi
            mid = jnp.minimum((lo + hi) // 2, NB16 - 1)
            pltpu.sync_copy(ids_hbm.at[pl.ds(mid * 16, 16)], bs_v)
            first = bs_v[...][0]
            ge = first >= v
            new_lo = jnp.where(done, lo, jnp.where(ge, lo, mid + 1))
            new_hi = jnp.where(done, hi, jnp.where(ge, mid, hi))
            return (new_lo, new_hi)

        fb, _ = lax.fori_loop(0, 15, body, (jnp.int32(0), jnp.int32(NB16)))
        blk = jnp.maximum(fb - 1, 0)
        pltpu.sync_copy(ids_hbm.at[pl.ds(blk * 16, 16)], bs_v)
        vec = bs_v[...]
        cnt_lt = jnp.int32(0)
        for j in range(16):
            cnt_lt = cnt_lt + jnp.where(vec[j] < v, 1, 0)
        return jnp.where(fb == 0, 0, (fb - 1) * 16 + cnt_lt)

    r0 = lower_bound(lo_seg)
    r1 = lower_bound(lo_seg + SEG_PER_W)
    base = (r0 // CHUNK) * CHUNK
    nchunks = (r1 - base + CHUNK - 1) // CHUNK
    npairs = (nchunks + 1) // 2

    def chunk_start(ci):
        return jnp.minimum(base + ci * CHUNK, N - CHUNK)

    def issue(ci, ids_vb, h_vb, semb):
        st = chunk_start(ci)
        pltpu.async_copy(ids_hbm.at[pl.ds(st, CHUNK)], ids_vb, semb)
        pltpu.async_copy(h_hbm.at[pl.ds(st * D_H, CHUNK * D_H)], h_vb, semb)

    def drain(ci, ids_vb, h_vb, semb):
        st = chunk_start(ci)
        pltpu.make_async_copy(ids_hbm.at[pl.ds(st, CHUNK)], ids_vb, semb).wait()
        pltpu.make_async_copy(h_hbm.at[pl.ds(st * D_H, CHUNK * D_H)], h_vb, semb).wait()

    def neutralize(ci, ids_vb):
        @pl.when(ci >= nchunks)
        def _():
            for g in range(CHUNK // 16):
                ids_vb[pl.ds(g * 16, 16)] = neg16

    def flush(rel, cnt, a, m):
        for j in range(4):
            sum_v[pl.ds(rel * D_H + j * 16, 16)] = a[j]
            mean_v[pl.ds(rel * D_H + j * 16, 16)] = a[j] / cnt
            max_v[pl.ds(rel * D_H + j * 16, 16)] = m[j]

    def walk(ids_vb, h_vb, carry):
        def row_step(i, idv, rc):
            cur_rel, cnt, a0, a1, a2, a3, m0, m1, m2, m3 = rc
            rel = idv - lo_seg
            valid = (rel >= 0) & (rel < SEG_PER_W)
            rel = jnp.where(valid, rel, SEG_PER_W)
            v0 = h_vb[pl.ds(i * D_H, 16)]
            v1 = h_vb[pl.ds(i * D_H + 16, 16)]
            v2 = h_vb[pl.ds(i * D_H + 32, 16)]
            v3 = h_vb[pl.ds(i * D_H + 48, 16)]
            is_new = rel != cur_rel

            @pl.when(is_new & (cur_rel >= 0))
            def _():
                flush(cur_rel, cnt, (a0, a1, a2, a3), (m0, m1, m2, m3))

            a0 = jnp.where(is_new, v0, a0 + v0)
            a1 = jnp.where(is_new, v1, a1 + v1)
            a2 = jnp.where(is_new, v2, a2 + v2)
            a3 = jnp.where(is_new, v3, a3 + v3)
            m0 = jnp.where(is_new, v0, jnp.maximum(m0, v0))
            m1 = jnp.where(is_new, v1, jnp.maximum(m1, v1))
            m2 = jnp.where(is_new, v2, jnp.maximum(m2, v2))
            m3 = jnp.where(is_new, v3, jnp.maximum(m3, v3))
            cnt = jnp.where(is_new, 1.0, cnt + 1.0)
            return (rel, cnt, a0, a1, a2, a3, m0, m1, m2, m3)

        def group_body(g, rc):
            idvec = ids_vb[pl.ds(g * 16, 16)]
            for j in range(16):
                rc = row_step(g * 16 + j, idvec[j], rc)
            return rc

        return lax.fori_loop(0, CHUNK // 16, group_body, carry)

    issue(jnp.int32(0), ids_v0, h_v0, sem0)

    def pair_body(k, carry):
        ci0 = 2 * k
        issue(ci0 + 1, ids_v1, h_v1, sem1)
        drain(ci0, ids_v0, h_v0, sem0)
        neutralize(ci0, ids_v0)
        carry = walk(ids_v0, h_v0, carry)
        issue(ci0 + 2, ids_v0, h_v0, sem0)
        drain(ci0 + 1, ids_v1, h_v1, sem1)
        neutralize(ci0 + 1, ids_v1)
        carry = walk(ids_v1, h_v1, carry)
        return carry

    init = (jnp.int32(-1), jnp.float32(0.0), z16, z16, z16, z16, z16, z16, z16, z16)
    fin = lax.fori_loop(0, npairs, pair_body, init)
    drain(2 * npairs, ids_v0, h_v0, sem0)
    cur_rel, cnt = fin[0], fin[1]

    @pl.when(cur_rel >= 0)
    def _():
        flush(cur_rel, cnt, fin[2:6], fin[6:10])

    pltpu.sync_copy(sum_v.at[pl.ds(0, SEG_PER_W * D_H)],
                    sum_hbm.at[pl.ds(lo_seg * D_H, SEG_PER_W * D_H)])
    pltpu.sync_copy(mean_v.at[pl.ds(0, SEG_PER_W * D_H)],
                    mean_hbm.at[pl.ds(lo_seg * D_H, SEG_PER_W * D_H)])
    pltpu.sync_copy(max_v.at[pl.ds(0, SEG_PER_W * D_H)],
                    max_hbm.at[pl.ds(lo_seg * D_H, SEG_PER_W * D_H)])


def _segstats(ids, h_flat):
    mesh = plsc.VectorSubcoreMesh(core_axis_name="c", subcore_axis_name="s",
                                  num_cores=NC, num_subcores=NS)
    f = pl.kernel(
        _seg_body,
        out_type=[jax.ShapeDtypeStruct((SP * D_H,), jnp.float32)] * 3,
        mesh=mesh,
        scratch_types=[
            pltpu.VMEM(((SEG_PER_W + 1) * D_H,), jnp.float32),
            pltpu.VMEM(((SEG_PER_W + 1) * D_H,), jnp.float32),
            pltpu.VMEM(((SEG_PER_W + 1) * D_H,), jnp.float32),
            pltpu.VMEM((CHUNK,), jnp.int32),
            pltpu.VMEM((CHUNK,), jnp.int32),
            pltpu.VMEM((CHUNK * D_H,), jnp.float32),
            pltpu.VMEM((CHUNK * D_H,), jnp.float32),
            pltpu.VMEM((16,), jnp.int32),
            pltpu.SemaphoreType.DMA,
            pltpu.SemaphoreType.DMA,
        ],
    )
    return f(ids, h_flat)


def _rho_body(s_ref, mn_ref, mx_ref, a_ref, b_ref, c_ref, br1_ref, w2_ref,
              br2_ref, o_ref):
    y = lax.dot_general(s_ref[...], a_ref[...], _DN, preferred_element_type=jnp.float32)
    y = y + lax.dot_general(mn_ref[...], b_ref[...], _DN, preferred_element_type=jnp.float32)
    y = y + lax.dot_general(mx_ref[...], c_ref[...], _DN, preferred_element_type=jnp.float32)
    y = y + br1_ref[...]
    z = lax.dot_general(y, w2_ref[...], _DN, preferred_element_type=jnp.float32)
    z = z + br2_ref[...]
    zmax = jnp.max(z, axis=1, keepdims=True)
    zs = z - zmax
    lse = jnp.log(jnp.sum(jnp.exp(zs), axis=1, keepdims=True))
    o_ref[...] = zs - lse


def _rho(xs, xmn, xmx, wa, wb, wc, br1, w2, br2):
    blk = 1000
    grid = (S // blk,)
    return pl.pallas_call(
        _rho_body,
        grid=grid,
        in_specs=[
            pl.BlockSpec((blk, D_H), lambda i: (i, 0)),
            pl.BlockSpec((blk, D_H), lambda i: (i, 0)),
            pl.BlockSpec((blk, D_H), lambda i: (i, 0)),
            pl.BlockSpec((D_H, D_H), lambda i: (0, 0)),
            pl.BlockSpec((D_H, D_H), lambda i: (0, 0)),
            pl.BlockSpec((D_H, D_H), lambda i: (0, 0)),
            pl.BlockSpec((1, D_H), lambda i: (0, 0)),
            pl.BlockSpec((10, D_H), lambda i: (0, 0)),
            pl.BlockSpec((1, 10), lambda i: (0, 0)),
        ],
        out_specs=pl.BlockSpec((blk, 10), lambda i: (i, 0)),
        out_shape=jax.ShapeDtypeStruct((S, 10), jnp.float32),
        compiler_params=pltpu.CompilerParams(dimension_semantics=("parallel",)),
    )(xs, xmn, xmx, wa, wb, wc, br1, w2, br2)


def kernel(x, l, W_p1, b_p1, W_p2, b_p2, W_r1, b_r1, W_r2, b_r2):
    ids = l[:, 0].astype(jnp.int32)
    h = _phi(x, W_p1, b_p1.reshape(1, -1), W_p2, b_p2.reshape(1, -1))
    sums, means, maxs = _segstats(ids, h.reshape(-1))
    sums = sums.reshape(SP, D_H)[:S]
    means = means.reshape(SP, D_H)[:S]
    maxs = maxs.reshape(SP, D_H)[:S]
    out = _rho(sums, means, maxs,
               W_r1[:, 0:64], W_r1[:, 64:128], W_r1[:, 128:192],
               b_r1.reshape(1, -1), W_r2, b_r2.reshape(1, -1))
    return out


# speedup vs baseline: 5.6439x; 1.3438x over previous
import functools

import jax
import jax.numpy as jnp
from jax import lax
from jax.experimental import pallas as pl
from jax.experimental.pallas import tpu as pltpu
from jax.experimental.pallas import tpu_sc as plsc

N = 320000
D_IN = 128
D_H = 64
S = 10000

NC, NS = 2, 16
NW = NC * NS
SEG_PER_W = 320
SP = SEG_PER_W * NW
CHUNK = 256
NB16 = N // 16
RS = 2 * D_H

_DN = (((1,), (1,)), ((), ()))


def _phi_body(x_ref, w1_ref, b1_ref, w2_ref, b2_ref, h_ref):
    x = x_ref[...]
    h1 = lax.dot_general(x, w1_ref[...], _DN, preferred_element_type=jnp.float32)
    h1 = jnp.maximum(h1 + b1_ref[...], 0.0)
    h2 = lax.dot_general(h1, w2_ref[...], _DN, preferred_element_type=jnp.float32)
    h_ref[:, 0:D_H] = jnp.maximum(h2 + b2_ref[...], 0.0)


def _phi(x, w1, b1, w2, b2):
    blk = 2560
    grid = (N // blk,)
    return pl.pallas_call(
        _phi_body,
        grid=grid,
        in_specs=[
            pl.BlockSpec((blk, D_IN), lambda i: (i, 0)),
            pl.BlockSpec((D_IN, D_IN), lambda i: (0, 0)),
            pl.BlockSpec((1, D_IN), lambda i: (0, 0)),
            pl.BlockSpec((D_H, D_IN), lambda i: (0, 0)),
            pl.BlockSpec((1, D_H), lambda i: (0, 0)),
        ],
        out_specs=pl.BlockSpec((blk, RS), lambda i: (i, 0)),
        out_shape=jax.ShapeDtypeStruct((N, RS), jnp.float32),
        compiler_params=pltpu.CompilerParams(dimension_semantics=("parallel",)),
    )(x, w1, b1, w2, b2)


def _seg_body(ids_hbm, h_hbm, sum_hbm, mean_hbm, max_hbm,
             sum_v, mean_v, max_v, ids_v0, ids_v1, h_v0, h_v1, bs_v,
             sem0, sem1):
    c = lax.axis_index("c")
    s = lax.axis_index("s")
    wid = s * NC + c
    lo_seg = wid * SEG_PER_W

    z16 = jnp.zeros((16,), jnp.float32)
    neg16 = jnp.full((16,), -1, jnp.int32)

    def zrow(i, carry):
        sum_v[pl.ds(i * 16, 16)] = z16
        mean_v[pl.ds(i * 16, 16)] = z16
        max_v[pl.ds(i * 16, 16)] = z16
        return carry

    lax.fori_loop(0, SEG_PER_W * 4, zrow, 0)

    def lower_bound(v):
        def body(_, state):
            lo, hi = state
            done = lo >= hi
            mid = jnp.minimum((lo + hi) // 2, NB16 - 1)
            pltpu.sync_copy(ids_hbm.at[pl.ds(mid * 16, 16)], bs_v)
            first = bs_v[...][0]
            ge = first >= v
            new_lo = jnp.where(done, lo, jnp.where(ge, lo, mid + 1))
            new_hi = jnp.where(done, hi, jnp.where(ge, mid, hi))
            return (new_lo, new_hi)

        fb, _ = lax.fori_loop(0, 15, body, (jnp.int32(0), jnp.int32(NB16)))
        blk = jnp.maximum(fb - 1, 0)
        pltpu.sync_copy(ids_hbm.at[pl.ds(blk * 16, 16)], bs_v)
        vec = bs_v[...]
        cnt_lt = jnp.int32(0)
        for j in range(16):
            cnt_lt = cnt_lt + jnp.where(vec[j] < v, 1, 0)
        return jnp.where(fb == 0, 0, (fb - 1) * 16 + cnt_lt)

    r0 = lower_bound(lo_seg)
    r1 = lower_bound(lo_seg + SEG_PER_W)
    base = (r0 // CHUNK) * CHUNK
    nchunks = (r1 - base + CHUNK - 1) // CHUNK
    npairs = (nchunks + 1) // 2

    def chunk_start(ci):
        return jnp.minimum(base + ci * CHUNK, N - CHUNK)

    def issue(ci, ids_vb, h_vb, semb):
        st = chunk_start(ci)
        pltpu.async_copy(ids_hbm.at[pl.ds(st, CHUNK)], ids_vb, semb)
        pltpu.async_copy(h_hbm.at[pl.ds(st * RS, CHUNK * RS)], h_vb, semb)

    def drain(ci, ids_vb, h_vb, semb):
        st = chunk_start(ci)
        pltpu.make_async_copy(ids_hbm.at[pl.ds(st, CHUNK)], ids_vb, semb).wait()
        pltpu.make_async_copy(h_hbm.at[pl.ds(st * RS, CHUNK * RS)], h_vb, semb).wait()

    def neutralize(ci, ids_vb):
        @pl.when(ci >= nchunks)
        def _():
            for g in range(CHUNK // 16):
                ids_vb[pl.ds(g * 16, 16)] = neg16

    def flush(rel, cnt, a, m):
        for j in range(4):
            sum_v[pl.ds(rel * D_H + j * 16, 16)] = a[j]
            mean_v[pl.ds(rel * D_H + j * 16, 16)] = a[j] / cnt
            max_v[pl.ds(rel * D_H + j * 16, 16)] = m[j]

    def walk(ids_vb, h_vb, carry):
        def row_step(i, idv, rc):
            cur_rel, cnt, a0, a1, a2, a3, m0, m1, m2, m3 = rc
            rel = idv - lo_seg
            valid = (rel >= 0) & (rel < SEG_PER_W)
            rel = jnp.where(valid, rel, SEG_PER_W)
            v0 = h_vb[pl.ds(i * RS, 16)]
            v1 = h_vb[pl.ds(i * RS + 16, 16)]
            v2 = h_vb[pl.ds(i * RS + 32, 16)]
            v3 = h_vb[pl.ds(i * RS + 48, 16)]
            is_new = rel != cur_rel

            @pl.when(is_new & (cur_rel >= 0))
            def _():
                flush(cur_rel, cnt, (a0, a1, a2, a3), (m0, m1, m2, m3))

            a0 = jnp.where(is_new, v0, a0 + v0)
            a1 = jnp.where(is_new, v1, a1 + v1)
            a2 = jnp.where(is_new, v2, a2 + v2)
            a3 = jnp.where(is_new, v3, a3 + v3)
            m0 = jnp.where(is_new, v0, jnp.maximum(m0, v0))
            m1 = jnp.where(is_new, v1, jnp.maximum(m1, v1))
            m2 = jnp.where(is_new, v2, jnp.maximum(m2, v2))
            m3 = jnp.where(is_new, v3, jnp.maximum(m3, v3))
            cnt = jnp.where(is_new, 1.0, cnt + 1.0)
            return (rel, cnt, a0, a1, a2, a3, m0, m1, m2, m3)

        def group_body(g, rc):
            idvec = ids_vb[pl.ds(g * 16, 16)]
            for j in range(16):
                rc = row_step(g * 16 + j, idvec[j], rc)
            return rc

        return lax.fori_loop(0, CHUNK // 16, group_body, carry)

    issue(jnp.int32(0), ids_v0, h_v0, sem0)

    def pair_body(k, carry):
        ci0 = 2 * k
        issue(ci0 + 1, ids_v1, h_v1, sem1)
        drain(ci0, ids_v0, h_v0, sem0)
        neutralize(ci0, ids_v0)
        carry = walk(ids_v0, h_v0, carry)
        issue(ci0 + 2, ids_v0, h_v0, sem0)
        drain(ci0 + 1, ids_v1, h_v1, sem1)
        neutralize(ci0 + 1, ids_v1)
        carry = walk(ids_v1, h_v1, carry)
        return carry

    init = (jnp.int32(-1), jnp.float32(0.0), z16, z16, z16, z16, z16, z16, z16, z16)
    fin = lax.fori_loop(0, npairs, pair_body, init)
    drain(2 * npairs, ids_v0, h_v0, sem0)
    cur_rel, cnt = fin[0], fin[1]

    @pl.when(cur_rel >= 0)
    def _():
        flush(cur_rel, cnt, fin[2:6], fin[6:10])

    pltpu.sync_copy(sum_v.at[pl.ds(0, SEG_PER_W * D_H)],
                    sum_hbm.at[pl.ds(lo_seg * D_H, SEG_PER_W * D_H)])
    pltpu.sync_copy(mean_v.at[pl.ds(0, SEG_PER_W * D_H)],
                    mean_hbm.at[pl.ds(lo_seg * D_H, SEG_PER_W * D_H)])
    pltpu.sync_copy(max_v.at[pl.ds(0, SEG_PER_W * D_H)],
                    max_hbm.at[pl.ds(lo_seg * D_H, SEG_PER_W * D_H)])


def _segstats(ids, h_flat):
    mesh = plsc.VectorSubcoreMesh(core_axis_name="c", subcore_axis_name="s",
                                  num_cores=NC, num_subcores=NS)
    f = pl.kernel(
        _seg_body,
        out_type=[jax.ShapeDtypeStruct((SP * D_H,), jnp.float32)] * 3,
        mesh=mesh,
        scratch_types=[
            pltpu.VMEM(((SEG_PER_W + 1) * D_H,), jnp.float32),
            pltpu.VMEM(((SEG_PER_W + 1) * D_H,), jnp.float32),
            pltpu.VMEM(((SEG_PER_W + 1) * D_H,), jnp.float32),
            pltpu.VMEM((CHUNK,), jnp.int32),
            pltpu.VMEM((CHUNK,), jnp.int32),
            pltpu.VMEM((CHUNK * RS,), jnp.float32),
            pltpu.VMEM((CHUNK * RS,), jnp.float32),
            pltpu.VMEM((16,), jnp.int32),
            pltpu.SemaphoreType.DMA,
            pltpu.SemaphoreType.DMA,
        ],
    )
    return f(ids, h_flat)


def _rho_body(s_ref, mn_ref, mx_ref, a_ref, b_ref, c_ref, br1_ref, w2_ref,
              br2_ref, o_ref):
    y = lax.dot_general(s_ref[...], a_ref[...], _DN, preferred_element_type=jnp.float32)
    y = y + lax.dot_general(mn_ref[...], b_ref[...], _DN, preferred_element_type=jnp.float32)
    y = y + lax.dot_general(mx_ref[...], c_ref[...], _DN, preferred_element_type=jnp.float32)
    y = y + br1_ref[...]
    z = lax.dot_general(y, w2_ref[...], _DN, preferred_element_type=jnp.float32)
    z = z + br2_ref[...]
    zmax = jnp.max(z, axis=1, keepdims=True)
    zs = z - zmax
    lse = jnp.log(jnp.sum(jnp.exp(zs), axis=1, keepdims=True))
    o_ref[...] = zs - lse


def _rho(xs, xmn, xmx, wa, wb, wc, br1, w2, br2):
    blk = 1000
    grid = (S // blk,)
    return pl.pallas_call(
        _rho_body,
        grid=grid,
        in_specs=[
            pl.BlockSpec((blk, D_H), lambda i: (i, 0)),
            pl.BlockSpec((blk, D_H), lambda i: (i, 0)),
            pl.BlockSpec((blk, D_H), lambda i: (i, 0)),
            pl.BlockSpec((D_H, D_H), lambda i: (0, 0)),
            pl.BlockSpec((D_H, D_H), lambda i: (0, 0)),
            pl.BlockSpec((D_H, D_H), lambda i: (0, 0)),
            pl.BlockSpec((1, D_H), lambda i: (0, 0)),
            pl.BlockSpec((10, D_H), lambda i: (0, 0)),
            pl.BlockSpec((1, 10), lambda i: (0, 0)),
        ],
        out_specs=pl.BlockSpec((blk, 10), lambda i: (i, 0)),
        out_shape=jax.ShapeDtypeStruct((S, 10), jnp.float32),
        compiler_params=pltpu.CompilerParams(dimension_semantics=("parallel",)),
    )(xs, xmn, xmx, wa, wb, wc, br1, w2, br2)


def kernel(x, l, W_p1, b_p1, W_p2, b_p2, W_r1, b_r1, W_r2, b_r2):
    ids = l[:, 0].astype(jnp.int32)
    h = _phi(x, W_p1, b_p1.reshape(1, -1), W_p2, b_p2.reshape(1, -1))
    sums, means, maxs = _segstats(ids, h.reshape(-1))
    sums = sums.reshape(SP, D_H)[:S]
    means = means.reshape(SP, D_H)[:S]
    maxs = maxs.reshape(SP, D_H)[:S]
    out = _rho(sums, means, maxs,
               W_r1[:, 0:64], W_r1[:, 64:128], W_r1[:, 128:192],
               b_r1.reshape(1, -1), W_r2, b_r2.reshape(1, -1))
    return out


# speedup vs baseline: 6.0018x; 1.0634x over previous
import functools

import jax
import jax.numpy as jnp
from jax import lax
from jax.experimental import pallas as pl
from jax.experimental.pallas import tpu as pltpu
from jax.experimental.pallas import tpu_sc as plsc

N = 320000
D_IN = 128
D_H = 64
S = 10000

NC, NS = 2, 16
NW = NC * NS
SEG_PER_W = 320
SP = SEG_PER_W * NW
CHUNK = 256
NB16 = N // 16
RS = 2 * D_H

_DN = (((1,), (1,)), ((), ()))


def _phi_body(x_ref, w1_ref, b1_ref, w2_ref, b2_ref, h_ref):
    x = x_ref[...]
    h1 = lax.dot_general(x, w1_ref[...], _DN, preferred_element_type=jnp.float32)
    h1 = jnp.maximum(h1 + b1_ref[...], 0.0)
    h2 = lax.dot_general(h1, w2_ref[...], _DN, preferred_element_type=jnp.float32)
    h_ref[:, 0:D_H] = jnp.maximum(h2 + b2_ref[...], 0.0)


def _phi(x, w1, b1, w2, b2):
    blk = 2560
    grid = (N // blk,)
    return pl.pallas_call(
        _phi_body,
        grid=grid,
        in_specs=[
            pl.BlockSpec((blk, D_IN), lambda i: (i, 0)),
            pl.BlockSpec((D_IN, D_IN), lambda i: (0, 0)),
            pl.BlockSpec((1, D_IN), lambda i: (0, 0)),
            pl.BlockSpec((D_H, D_IN), lambda i: (0, 0)),
            pl.BlockSpec((1, D_H), lambda i: (0, 0)),
        ],
        out_specs=pl.BlockSpec((blk, RS), lambda i: (i, 0)),
        out_shape=jax.ShapeDtypeStruct((N, RS), jnp.float32),
        compiler_params=pltpu.CompilerParams(dimension_semantics=("parallel",)),
    )(x, w1, b1, w2, b2)


def _seg_body(ids_hbm, h_hbm, sum_hbm, mean_hbm, max_hbm,
             sum_v, mean_v, max_v, ids_v0, ids_v1, h_v0, h_v1, bs_v,
             riv, cv, sem0, sem1):
    c = lax.axis_index("c")
    s = lax.axis_index("s")
    wid = s * NC + c
    lo_seg = wid * SEG_PER_W

    z16 = jnp.zeros((16,), jnp.float32)
    neg16 = jnp.full((16,), -1, jnp.int32)

    def zrow(i, carry):
        sum_v[pl.ds(i * 16, 16)] = z16
        mean_v[pl.ds(i * 16, 16)] = z16
        max_v[pl.ds(i * 16, 16)] = z16
        return carry

    lax.fori_loop(0, SEG_PER_W * 4, zrow, 0)

    riv[pl.ds(0, 16)] = jnp.full((16,), -1, jnp.int32)
    for k in range(9):
        cv[pl.ds(k * 16, 16)] = z16

    def lower_bound(v):
        def body(_, state):
            lo, hi = state
            done = lo >= hi
            mid = jnp.minimum((lo + hi) // 2, NB16 - 1)
            pltpu.sync_copy(ids_hbm.at[pl.ds(mid * 16, 16)], bs_v)
            first = bs_v[...][0]
            ge = first >= v
            new_lo = jnp.where(done, lo, jnp.where(ge, lo, mid + 1))
            new_hi = jnp.where(done, hi, jnp.where(ge, mid, hi))
            return (new_lo, new_hi)

        fb, _ = lax.fori_loop(0, 15, body, (jnp.int32(0), jnp.int32(NB16)))
        blk = jnp.maximum(fb - 1, 0)
        pltpu.sync_copy(ids_hbm.at[pl.ds(blk * 16, 16)], bs_v)
        vec = bs_v[...]
        cnt_lt = jnp.int32(0)
        for j in range(16):
            cnt_lt = cnt_lt + jnp.where(vec[j] < v, 1, 0)
        return jnp.where(fb == 0, 0, (fb - 1) * 16 + cnt_lt)

    r0 = lower_bound(lo_seg)
    r1 = lower_bound(lo_seg + SEG_PER_W)
    base = (r0 // CHUNK) * CHUNK
    nchunks = (r1 - base + CHUNK - 1) // CHUNK
    npairs = (nchunks + 1) // 2

    def chunk_start(ci):
        return jnp.minimum(base + ci * CHUNK, N - CHUNK)

    def issue(ci, ids_vb, h_vb, semb):
        st = chunk_start(ci)
        pltpu.async_copy(ids_hbm.at[pl.ds(st, CHUNK)], ids_vb, semb)
        pltpu.async_copy(h_hbm.at[pl.ds(st * RS, CHUNK * RS)], h_vb, semb)

    def drain(ci, ids_vb, h_vb, semb):
        st = chunk_start(ci)
        pltpu.make_async_copy(ids_hbm.at[pl.ds(st, CHUNK)], ids_vb, semb).wait()
        pltpu.make_async_copy(h_hbm.at[pl.ds(st * RS, CHUNK * RS)], h_vb, semb).wait()

    def neutralize(ci, ids_vb):
        @pl.when(ci >= nchunks)
        def _():
            for g in range(CHUNK // 16):
                ids_vb[pl.ds(g * 16, 16)] = neg16

    def flush(rel, cnt, a, m):
        for j in range(4):
            sum_v[pl.ds(rel * D_H + j * 16, 16)] = a[j]
            mean_v[pl.ds(rel * D_H + j * 16, 16)] = a[j] / cnt
            max_v[pl.ds(rel * D_H + j * 16, 16)] = m[j]

    def walk(ids_vb, h_vb):
        def row_step(i, idv, rc):
            cur_rel, cnt, a0, a1, a2, a3, m0, m1, m2, m3 = rc
            rel = idv - lo_seg
            valid = (rel >= 0) & (rel < SEG_PER_W)
            rel = jnp.where(valid, rel, SEG_PER_W)
            v0 = h_vb[pl.ds(i * RS, 16)]
            v1 = h_vb[pl.ds(i * RS + 16, 16)]
            v2 = h_vb[pl.ds(i * RS + 32, 16)]
            v3 = h_vb[pl.ds(i * RS + 48, 16)]
            is_new = rel != cur_rel

            @pl.when(is_new & (cur_rel >= 0))
            def _():
                flush(cur_rel, cnt, (a0, a1, a2, a3), (m0, m1, m2, m3))

            a0 = jnp.where(is_new, v0, a0 + v0)
            a1 = jnp.where(is_new, v1, a1 + v1)
            a2 = jnp.where(is_new, v2, a2 + v2)
            a3 = jnp.where(is_new, v3, a3 + v3)
            m0 = jnp.where(is_new, v0, jnp.maximum(m0, v0))
            m1 = jnp.where(is_new, v1, jnp.maximum(m1, v1))
            m2 = jnp.where(is_new, v2, jnp.maximum(m2, v2))
            m3 = jnp.where(is_new, v3, jnp.maximum(m3, v3))
            cnt = jnp.where(is_new, 1.0, cnt + 1.0)
            return (rel, cnt, a0, a1, a2, a3, m0, m1, m2, m3)

        def group_body(g, dummy):
            idvec = ids_vb[pl.ds(g * 16, 16)]
            e0 = idvec[0]
            e15 = idvec[15]
            cur_rel = riv[pl.ds(0, 16)][0]
            cnt = cv[pl.ds(0, 16)][0]
            a = [cv[pl.ds((1 + k) * 16, 16)] for k in range(4)]
            m = [cv[pl.ds((5 + k) * 16, 16)] for k in range(4)]

            def save(rel, ncnt, na, nm):
                riv[pl.ds(0, 16)] = jnp.full((16,), 1, jnp.int32) * rel
                cv[pl.ds(0, 16)] = jnp.full((16,), 1.0, jnp.float32) * ncnt
                for k in range(4):
                    cv[pl.ds((1 + k) * 16, 16)] = na[k]
                    cv[pl.ds((5 + k) * 16, 16)] = nm[k]

            @pl.when(e0 == e15)
            def _():
                rel = e0 - lo_seg
                valid = (rel >= 0) & (rel < SEG_PER_W)
                rel = jnp.where(valid, rel, SEG_PER_W)
                ts = []
                tm = []
                for j in range(4):
                    vals = [h_vb[pl.ds((g * 16 + r) * RS + j * 16, 16)]
                            for r in range(16)]
                    sacc = vals
                    while len(sacc) > 1:
                        sacc = [sacc[k] + sacc[k + 1]
                                for k in range(0, len(sacc), 2)]
                    macc = vals
                    while len(macc) > 1:
                        macc = [jnp.maximum(macc[k], macc[k + 1])
                                for k in range(0, len(macc), 2)]
                    ts.append(sacc[0])
                    tm.append(macc[0])
                is_new = rel != cur_rel

                @pl.when(is_new & (cur_rel >= 0))
                def _():
                    flush(cur_rel, cnt, a, m)

                na = [jnp.where(is_new, ts[k], a[k] + ts[k]) for k in range(4)]
                nm = [jnp.where(is_new, tm[k], jnp.maximum(m[k], tm[k]))
                      for k in range(4)]
                ncnt = jnp.where(is_new, 16.0, cnt + 16.0)
                save(rel, ncnt, na, nm)

            @pl.when(e0 != e15)
            def _():
                rc = (cur_rel, cnt, a[0], a[1], a[2], a[3],
                      m[0], m[1], m[2], m[3])
                for j in range(16):
                    rc = row_step(g * 16 + j, idvec[j], rc)
                save(rc[0], rc[1], rc[2:6], rc[6:10])

            return dummy

        lax.fori_loop(0, CHUNK // 16, group_body, 0)

    issue(jnp.int32(0), ids_v0, h_v0, sem0)

    def pair_body(k, dummy):
        ci0 = 2 * k
        issue(ci0 + 1, ids_v1, h_v1, sem1)
        drain(ci0, ids_v0, h_v0, sem0)
        neutralize(ci0, ids_v0)
        walk(ids_v0, h_v0)
        issue(ci0 + 2, ids_v0, h_v0, sem0)
        drain(ci0 + 1, ids_v1, h_v1, sem1)
        neutralize(ci0 + 1, ids_v1)
        walk(ids_v1, h_v1)
        return dummy

    lax.fori_loop(0, npairs, pair_body, 0)
    drain(2 * npairs, ids_v0, h_v0, sem0)
    cur_rel = riv[pl.ds(0, 16)][0]
    cnt = cv[pl.ds(0, 16)][0]

    @pl.when(cur_rel >= 0)
    def _():
        flush(cur_rel, cnt,
              [cv[pl.ds((1 + k) * 16, 16)] for k in range(4)],
              [cv[pl.ds((5 + k) * 16, 16)] for k in range(4)])

    pltpu.sync_copy(sum_v.at[pl.ds(0, SEG_PER_W * D_H)],
                    sum_hbm.at[pl.ds(lo_seg * D_H, SEG_PER_W * D_H)])
    pltpu.sync_copy(mean_v.at[pl.ds(0, SEG_PER_W * D_H)],
                    mean_hbm.at[pl.ds(lo_seg * D_H, SEG_PER_W * D_H)])
    pltpu.sync_copy(max_v.at[pl.ds(0, SEG_PER_W * D_H)],
                    max_hbm.at[pl.ds(lo_seg * D_H, SEG_PER_W * D_H)])


def _segstats(ids, h_flat):
    mesh = plsc.VectorSubcoreMesh(core_axis_name="c", subcore_axis_name="s",
                                  num_cores=NC, num_subcores=NS)
    f = pl.kernel(
        _seg_body,
        out_type=[jax.ShapeDtypeStruct((SP * D_H,), jnp.float32)] * 3,
        mesh=mesh,
        scratch_types=[
            pltpu.VMEM(((SEG_PER_W + 1) * D_H,), jnp.float32),
            pltpu.VMEM(((SEG_PER_W + 1) * D_H,), jnp.float32),
            pltpu.VMEM(((SEG_PER_W + 1) * D_H,), jnp.float32),
            pltpu.VMEM((CHUNK,), jnp.int32),
            pltpu.VMEM((CHUNK,), jnp.int32),
            pltpu.VMEM((CHUNK * RS,), jnp.float32),
            pltpu.VMEM((CHUNK * RS,), jnp.float32),
            pltpu.VMEM((16,), jnp.int32),
            pltpu.VMEM((16,), jnp.int32),
            pltpu.VMEM((144,), jnp.float32),
            pltpu.SemaphoreType.DMA,
            pltpu.SemaphoreType.DMA,
        ],
    )
    return f(ids, h_flat)


def _rho_body(s_ref, mn_ref, mx_ref, a_ref, b_ref, c_ref, br1_ref, w2_ref,
              br2_ref, o_ref):
    y = lax.dot_general(s_ref[...], a_ref[...], _DN, preferred_element_type=jnp.float32)
    y = y + lax.dot_general(mn_ref[...], b_ref[...], _DN, preferred_element_type=jnp.float32)
    y = y + lax.dot_general(mx_ref[...], c_ref[...], _DN, preferred_element_type=jnp.float32)
    y = y + br1_ref[...]
    z = lax.dot_general(y, w2_ref[...], _DN, preferred_element_type=jnp.float32)
    z = z + br2_ref[...]
    zmax = jnp.max(z, axis=1, keepdims=True)
    zs = z - zmax
    lse = jnp.log(jnp.sum(jnp.exp(zs), axis=1, keepdims=True))
    o_ref[...] = zs - lse


def _rho(xs, xmn, xmx, wa, wb, wc, br1, w2, br2):
    blk = 1000
    grid = (S // blk,)
    return pl.pallas_call(
        _rho_body,
        grid=grid,
        in_specs=[
            pl.BlockSpec((blk, D_H), lambda i: (i, 0)),
            pl.BlockSpec((blk, D_H), lambda i: (i, 0)),
            pl.BlockSpec((blk, D_H), lambda i: (i, 0)),
            pl.BlockSpec((D_H, D_H), lambda i: (0, 0)),
            pl.BlockSpec((D_H, D_H), lambda i: (0, 0)),
            pl.BlockSpec((D_H, D_H), lambda i: (0, 0)),
            pl.BlockSpec((1, D_H), lambda i: (0, 0)),
            pl.BlockSpec((10, D_H), lambda i: (0, 0)),
            pl.BlockSpec((1, 10), lambda i: (0, 0)),
        ],
        out_specs=pl.BlockSpec((blk, 10), lambda i: (i, 0)),
        out_shape=jax.ShapeDtypeStruct((S, 10), jnp.float32),
        compiler_params=pltpu.CompilerParams(dimension_semantics=("parallel",)),
    )(xs, xmn, xmx, wa, wb, wc, br1, w2, br2)


def kernel(x, l, W_p1, b_p1, W_p2, b_p2, W_r1, b_r1, W_r2, b_r2):
    ids = l[:, 0].astype(jnp.int32)
    h = _phi(x, W_p1, b_p1.reshape(1, -1), W_p2, b_p2.reshape(1, -1))
    sums, means, maxs = _segstats(ids, h.reshape(-1))
    sums = sums.reshape(SP, D_H)[:S]
    means = means.reshape(SP, D_H)[:S]
    maxs = maxs.reshape(SP, D_H)[:S]
    out = _rho(sums, means, maxs,
               W_r1[:, 0:64], W_r1[:, 64:128], W_r1[:, 128:192],
               b_r1.reshape(1, -1), W_r2, b_r2.reshape(1, -1))
    return out


# speedup vs baseline: 6.0290x; 1.0045x over previous
import functools

import jax
import jax.numpy as jnp
from jax import lax
from jax.experimental import pallas as pl
from jax.experimental.pallas import tpu as pltpu
from jax.experimental.pallas import tpu_sc as plsc

N = 320000
D_IN = 128
D_H = 64
S = 10000

NC, NS = 2, 16
NW = NC * NS
SEG_PER_W = 320
SP = SEG_PER_W * NW
CHUNK = 256
NB16 = N // 16
RS = 2 * D_H

_DN = (((1,), (1,)), ((), ()))


def _phi_body(x_ref, w1_ref, b1_ref, w2_ref, b2_ref, h_ref):
    x = x_ref[...]
    h1 = lax.dot_general(x, w1_ref[...], _DN, preferred_element_type=jnp.float32)
    h1 = jnp.maximum(h1 + b1_ref[...], 0.0)
    h2 = lax.dot_general(h1, w2_ref[...], _DN, preferred_element_type=jnp.float32)
    h_ref[:, 0:D_H] = jnp.maximum(h2 + b2_ref[...], 0.0)


def _phi(x, w1, b1, w2, b2):
    blk = 2560
    grid = (N // blk,)
    return pl.pallas_call(
        _phi_body,
        grid=grid,
        in_specs=[
            pl.BlockSpec((blk, D_IN), lambda i: (i, 0)),
            pl.BlockSpec((D_IN, D_IN), lambda i: (0, 0)),
            pl.BlockSpec((1, D_IN), lambda i: (0, 0)),
            pl.BlockSpec((D_H, D_IN), lambda i: (0, 0)),
            pl.BlockSpec((1, D_H), lambda i: (0, 0)),
        ],
        out_specs=pl.BlockSpec((blk, RS), lambda i: (i, 0)),
        out_shape=jax.ShapeDtypeStruct((N, RS), jnp.float32),
        compiler_params=pltpu.CompilerParams(dimension_semantics=("parallel",)),
    )(x, w1, b1, w2, b2)


def _seg_body(ids_hbm, h_hbm, sum_hbm, mean_hbm, max_hbm,
             sum_v, mean_v, max_v, ids_v0, ids_v1, h_v0, h_v1, bs_v,
             riv, cv, sem0, sem1):
    c = lax.axis_index("c")
    s = lax.axis_index("s")
    wid = s * NC + c
    lo_seg = wid * SEG_PER_W

    z16 = jnp.zeros((16,), jnp.float32)
    neg16 = jnp.full((16,), -1, jnp.int32)

    def zrow(i, carry):
        sum_v[pl.ds(i * 16, 16)] = z16
        mean_v[pl.ds(i * 16, 16)] = z16
        max_v[pl.ds(i * 16, 16)] = z16
        return carry

    lax.fori_loop(0, SEG_PER_W * 4, zrow, 0)

    riv[pl.ds(0, 16)] = jnp.full((16,), -1, jnp.int32)
    for k in range(9):
        cv[pl.ds(k * 16, 16)] = z16

    def lower_bound(v):
        def body(_, state):
            lo, hi = state
            done = lo >= hi
            mid = jnp.minimum((lo + hi) // 2, NB16 - 1)
            pltpu.sync_copy(ids_hbm.at[pl.ds(mid * 16, 16)], bs_v)
            first = bs_v[...][0]
            ge = first >= v
            new_lo = jnp.where(done, lo, jnp.where(ge, lo, mid + 1))
            new_hi = jnp.where(done, hi, jnp.where(ge, mid, hi))
            return (new_lo, new_hi)

        fb, _ = lax.fori_loop(0, 15, body, (jnp.int32(0), jnp.int32(NB16)))
        blk = jnp.maximum(fb - 1, 0)
        pltpu.sync_copy(ids_hbm.at[pl.ds(blk * 16, 16)], bs_v)
        vec = bs_v[...]
        cnt_lt = jnp.int32(0)
        for j in range(16):
            cnt_lt = cnt_lt + jnp.where(vec[j] < v, 1, 0)
        return jnp.where(fb == 0, 0, (fb - 1) * 16 + cnt_lt)

    r0 = lower_bound(lo_seg)
    r1 = lower_bound(lo_seg + SEG_PER_W)
    base = (r0 // CHUNK) * CHUNK
    nchunks = (r1 - base + CHUNK - 1) // CHUNK
    npairs = (nchunks + 1) // 2

    def chunk_start(ci):
        return jnp.minimum(base + ci * CHUNK, N - CHUNK)

    def issue(ci, ids_vb, h_vb, semb):
        st = chunk_start(ci)
        pltpu.async_copy(ids_hbm.at[pl.ds(st, CHUNK)], ids_vb, semb)
        pltpu.async_copy(h_hbm.at[pl.ds(st * RS, CHUNK * RS)], h_vb, semb)

    def drain(ci, ids_vb, h_vb, semb):
        st = chunk_start(ci)
        pltpu.make_async_copy(ids_hbm.at[pl.ds(st, CHUNK)], ids_vb, semb).wait()
        pltpu.make_async_copy(h_hbm.at[pl.ds(st * RS, CHUNK * RS)], h_vb, semb).wait()

    def neutralize(ci, ids_vb):
        @pl.when(ci >= nchunks)
        def _():
            for g in range(CHUNK // 16):
                ids_vb[pl.ds(g * 16, 16)] = neg16

    def flush(rel, cnt, a, m):
        for j in range(4):
            sum_v[pl.ds(rel * D_H + j * 16, 16)] = a[j]
            mean_v[pl.ds(rel * D_H + j * 16, 16)] = a[j] / cnt
            max_v[pl.ds(rel * D_H + j * 16, 16)] = m[j]

    def walk(ids_vb, h_vb):
        def row_step(i, idv, rc):
            cur_rel, cnt, a0, a1, a2, a3, m0, m1, m2, m3 = rc
            rel = idv - lo_seg
            valid = (rel >= 0) & (rel < SEG_PER_W)
            rel = jnp.where(valid, rel, SEG_PER_W)
            v0 = h_vb[pl.ds(i * RS, 16)]
            v1 = h_vb[pl.ds(i * RS + 16, 16)]
            v2 = h_vb[pl.ds(i * RS + 32, 16)]
            v3 = h_vb[pl.ds(i * RS + 48, 16)]
            is_new = rel != cur_rel

            @pl.when(is_new & (cur_rel >= 0))
            def _():
                flush(cur_rel, cnt, (a0, a1, a2, a3), (m0, m1, m2, m3))

            a0 = jnp.where(is_new, v0, a0 + v0)
            a1 = jnp.where(is_new, v1, a1 + v1)
            a2 = jnp.where(is_new, v2, a2 + v2)
            a3 = jnp.where(is_new, v3, a3 + v3)
            m0 = jnp.where(is_new, v0, jnp.maximum(m0, v0))
            m1 = jnp.where(is_new, v1, jnp.maximum(m1, v1))
            m2 = jnp.where(is_new, v2, jnp.maximum(m2, v2))
            m3 = jnp.where(is_new, v3, jnp.maximum(m3, v3))
            cnt = jnp.where(is_new, 1.0, cnt + 1.0)
            return (rel, cnt, a0, a1, a2, a3, m0, m1, m2, m3)

        def group_body(g, dummy):
            idvec = ids_vb[pl.ds(g * 16, 16)]
            e0 = idvec[0]
            e15 = idvec[15]
            cur_rel = riv[pl.ds(0, 16)][0]
            cnt = cv[pl.ds(0, 16)][0]
            a = [cv[pl.ds((1 + k) * 16, 16)] for k in range(4)]
            m = [cv[pl.ds((5 + k) * 16, 16)] for k in range(4)]

            def save(rel, ncnt, na, nm):
                riv[pl.ds(0, 16)] = jnp.full((16,), 1, jnp.int32) * rel
                cv[pl.ds(0, 16)] = jnp.full((16,), 1.0, jnp.float32) * ncnt
                for k in range(4):
                    cv[pl.ds((1 + k) * 16, 16)] = na[k]
                    cv[pl.ds((5 + k) * 16, 16)] = nm[k]

            def reduce_run(base, rows, eid, cur_rel, cnt, a, m):
                rel = eid - lo_seg
                valid = (rel >= 0) & (rel < SEG_PER_W)
                rel = jnp.where(valid, rel, SEG_PER_W)
                ts = []
                tm = []
                for j in range(4):
                    vals = [h_vb[pl.ds((base + r) * RS + j * 16, 16)]
                            for r in range(rows)]
                    sacc = vals
                    while len(sacc) > 1:
                        sacc = [sacc[k] + sacc[k + 1]
                                for k in range(0, len(sacc), 2)]
                    macc = vals
                    while len(macc) > 1:
                        macc = [jnp.maximum(macc[k], macc[k + 1])
                                for k in range(0, len(macc), 2)]
                    ts.append(sacc[0])
                    tm.append(macc[0])
                is_new = rel != cur_rel

                @pl.when(is_new & (cur_rel >= 0))
                def _():
                    flush(cur_rel, cnt, a, m)

                na = [jnp.where(is_new, ts[k], a[k] + ts[k]) for k in range(4)]
                nm = [jnp.where(is_new, tm[k], jnp.maximum(m[k], tm[k]))
                      for k in range(4)]
                ncnt = jnp.where(is_new, float(rows), cnt + float(rows))
                save(rel, ncnt, na, nm)

            def slow_run(lo, hi, cur_rel, cnt, a, m):
                rc = (cur_rel, cnt, a[0], a[1], a[2], a[3],
                      m[0], m[1], m[2], m[3])
                for j in range(lo, hi):
                    rc = row_step(g * 16 + j, idvec[j], rc)
                save(rc[0], rc[1], rc[2:6], rc[6:10])

            @pl.when(e0 == e15)
            def _():
                reduce_run(g * 16, 16, e0, cur_rel, cnt, a, m)

            @pl.when(e0 != e15)
            def _():
                e7 = idvec[7]
                e8 = idvec[8]

                @pl.when(e0 == e7)
                def _():
                    reduce_run(g * 16, 8, e0, cur_rel, cnt, a, m)

                @pl.when(e0 != e7)
                def _():
                    slow_run(0, 8, cur_rel, cnt, a, m)

                cur2 = riv[pl.ds(0, 16)][0]
                cnt2 = cv[pl.ds(0, 16)][0]
                a2 = [cv[pl.ds((1 + k) * 16, 16)] for k in range(4)]
                m2 = [cv[pl.ds((5 + k) * 16, 16)] for k in range(4)]

                @pl.when(e8 == e15)
                def _():
                    reduce_run(g * 16 + 8, 8, e8, cur2, cnt2, a2, m2)

                @pl.when(e8 != e15)
                def _():
                    slow_run(8, 16, cur2, cnt2, a2, m2)

            return dummy

        lax.fori_loop(0, CHUNK // 16, group_body, 0)

    issue(jnp.int32(0), ids_v0, h_v0, sem0)

    def pair_body(k, dummy):
        ci0 = 2 * k
        issue(ci0 + 1, ids_v1, h_v1, sem1)
        drain(ci0, ids_v0, h_v0, sem0)
        neutralize(ci0, ids_v0)
        walk(ids_v0, h_v0)
        issue(ci0 + 2, ids_v0, h_v0, sem0)
        drain(ci0 + 1, ids_v1, h_v1, sem1)
        neutralize(ci0 + 1, ids_v1)
        walk(ids_v1, h_v1)
        return dummy

    lax.fori_loop(0, npairs, pair_body, 0)
    drain(2 * npairs, ids_v0, h_v0, sem0)
    cur_rel = riv[pl.ds(0, 16)][0]
    cnt = cv[pl.ds(0, 16)][0]

    @pl.when(cur_rel >= 0)
    def _():
        flush(cur_rel, cnt,
              [cv[pl.ds((1 + k) * 16, 16)] for k in range(4)],
              [cv[pl.ds((5 + k) * 16, 16)] for k in range(4)])

    pltpu.sync_copy(sum_v.at[pl.ds(0, SEG_PER_W * D_H)],
                    sum_hbm.at[pl.ds(lo_seg * D_H, SEG_PER_W * D_H)])
    pltpu.sync_copy(mean_v.at[pl.ds(0, SEG_PER_W * D_H)],
                    mean_hbm.at[pl.ds(lo_seg * D_H, SEG_PER_W * D_H)])
    pltpu.sync_copy(max_v.at[pl.ds(0, SEG_PER_W * D_H)],
                    max_hbm.at[pl.ds(lo_seg * D_H, SEG_PER_W * D_H)])


def _segstats(ids, h_flat):
    mesh = plsc.VectorSubcoreMesh(core_axis_name="c", subcore_axis_name="s",
                                  num_cores=NC, num_subcores=NS)
    f = pl.kernel(
        _seg_body,
        out_type=[jax.ShapeDtypeStruct((SP * D_H,), jnp.float32)] * 3,
        mesh=mesh,
        scratch_types=[
            pltpu.VMEM(((SEG_PER_W + 1) * D_H,), jnp.float32),
            pltpu.VMEM(((SEG_PER_W + 1) * D_H,), jnp.float32),
            pltpu.VMEM(((SEG_PER_W + 1) * D_H,), jnp.float32),
            pltpu.VMEM((CHUNK,), jnp.int32),
            pltpu.VMEM((CHUNK,), jnp.int32),
            pltpu.VMEM((CHUNK * RS,), jnp.float32),
            pltpu.VMEM((CHUNK * RS,), jnp.float32),
            pltpu.VMEM((16,), jnp.int32),
            pltpu.VMEM((16,), jnp.int32),
            pltpu.VMEM((144,), jnp.float32),
            pltpu.SemaphoreType.DMA,
            pltpu.SemaphoreType.DMA,
        ],
    )
    return f(ids, h_flat)


def _rho_body(s_ref, mn_ref, mx_ref, a_ref, b_ref, c_ref, br1_ref, w2_ref,
              br2_ref, o_ref):
    y = lax.dot_general(s_ref[...], a_ref[...], _DN, preferred_element_type=jnp.float32)
    y = y + lax.dot_general(mn_ref[...], b_ref[...], _DN, preferred_element_type=jnp.float32)
    y = y + lax.dot_general(mx_ref[...], c_ref[...], _DN, preferred_element_type=jnp.float32)
    y = y + br1_ref[...]
    z = lax.dot_general(y, w2_ref[...], _DN, preferred_element_type=jnp.float32)
    z = z + br2_ref[...]
    zmax = jnp.max(z, axis=1, keepdims=True)
    zs = z - zmax
    lse = jnp.log(jnp.sum(jnp.exp(zs), axis=1, keepdims=True))
    o_ref[...] = zs - lse


def _rho(xs, xmn, xmx, wa, wb, wc, br1, w2, br2):
    blk = 1000
    grid = (S // blk,)
    return pl.pallas_call(
        _rho_body,
        grid=grid,
        in_specs=[
            pl.BlockSpec((blk, D_H), lambda i: (i, 0)),
            pl.BlockSpec((blk, D_H), lambda i: (i, 0)),
            pl.BlockSpec((blk, D_H), lambda i: (i, 0)),
            pl.BlockSpec((D_H, D_H), lambda i: (0, 0)),
            pl.BlockSpec((D_H, D_H), lambda i: (0, 0)),
            pl.BlockSpec((D_H, D_H), lambda i: (0, 0)),
            pl.BlockSpec((1, D_H), lambda i: (0, 0)),
            pl.BlockSpec((10, D_H), lambda i: (0, 0)),
            pl.BlockSpec((1, 10), lambda i: (0, 0)),
        ],
        out_specs=pl.BlockSpec((blk, 10), lambda i: (i, 0)),
        out_shape=jax.ShapeDtypeStruct((S, 10), jnp.float32),
        compiler_params=pltpu.CompilerParams(dimension_semantics=("parallel",)),
    )(xs, xmn, xmx, wa, wb, wc, br1, w2, br2)


def kernel(x, l, W_p1, b_p1, W_p2, b_p2, W_r1, b_r1, W_r2, b_r2):
    ids = l[:, 0].astype(jnp.int32)
    h = _phi(x, W_p1, b_p1.reshape(1, -1), W_p2, b_p2.reshape(1, -1))
    sums, means, maxs = _segstats(ids, h.reshape(-1))
    sums = sums.reshape(SP, D_H)[:S]
    means = means.reshape(SP, D_H)[:S]
    maxs = maxs.reshape(SP, D_H)[:S]
    out = _rho(sums, means, maxs,
               W_r1[:, 0:64], W_r1[:, 64:128], W_r1[:, 128:192],
               b_r1.reshape(1, -1), W_r2, b_r2.reshape(1, -1))
    return out
